# baseline (speedup 1.0000x reference)
import functools

import jax
import jax.numpy as jnp
from jax import lax
from jax.experimental import pallas as pl
from jax.experimental.pallas import tpu as pltpu
from jax.experimental.pallas import tpu_sc as plsc

N_DIM = 32
N_RATINGS = 5
B = 16384

_NC = 2
_NS = 16
_NW = _NC * _NS
_BPW = B // _NW


def _sc_gather_body(users_hbm, items_hbm, u_embT_hbm, i_embT_hbm,
                    uintT_hbm, iintT_hbm,
                    ueT_out, ieT_out, suT_out, siT_out,
                    idxu_v, idxi_v, ueT_v, ieT_v, suT_v, siT_v, sem):
    wid = lax.axis_index("s") * _NC + lax.axis_index("c")
    base = wid * _BPW
    pltpu.sync_copy(users_hbm.at[pl.ds(base, _BPW)], idxu_v)
    pltpu.sync_copy(items_hbm.at[pl.ds(base, _BPW)], idxi_v)
    copies = []
    for d in range(N_DIM):
        copies.append(pltpu.async_copy(
            u_embT_hbm.at[d].at[idxu_v], ueT_v.at[d], sem))
        copies.append(pltpu.async_copy(
            i_embT_hbm.at[d].at[idxi_v], ieT_v.at[d], sem))
    for d in range(N_RATINGS):
        copies.append(pltpu.async_copy(
            uintT_hbm.at[d].at[idxu_v], suT_v.at[d], sem))
        copies.append(pltpu.async_copy(
            iintT_hbm.at[d].at[idxi_v], siT_v.at[d], sem))
    for c in copies:
        c.wait()
    pltpu.sync_copy(ueT_v, ueT_out.at[:, pl.ds(base, _BPW)])
    pltpu.sync_copy(ieT_v, ieT_out.at[:, pl.ds(base, _BPW)])
    pltpu.sync_copy(suT_v, suT_out.at[:, pl.ds(base, _BPW)])
    pltpu.sync_copy(siT_v, siT_out.at[:, pl.ds(base, _BPW)])


_sc_gather = functools.partial(
    pl.kernel,
    out_type=(
        jax.ShapeDtypeStruct((N_DIM, B), jnp.float32),
        jax.ShapeDtypeStruct((N_DIM, B), jnp.float32),
        jax.ShapeDtypeStruct((N_RATINGS, B), jnp.float32),
        jax.ShapeDtypeStruct((N_RATINGS, B), jnp.float32),
    ),
    mesh=plsc.VectorSubcoreMesh(core_axis_name="c", subcore_axis_name="s"),
    compiler_params=pltpu.CompilerParams(use_tc_tiling_on_sc=False,
                                         needs_layout_passes=False),
    scratch_types=[
        pltpu.VMEM((_BPW,), jnp.int32),
        pltpu.VMEM((_BPW,), jnp.int32),
        pltpu.VMEM((N_DIM, _BPW), jnp.float32),
        pltpu.VMEM((N_DIM, _BPW), jnp.float32),
        pltpu.VMEM((N_RATINGS, _BPW), jnp.float32),
        pltpu.VMEM((N_RATINGS, _BPW), jnp.float32),
        pltpu.SemaphoreType.DMA,
    ],
)(_sc_gather_body)


def _tc_mlp_body(ueT_ref, ieT_ref, suT_ref, siT_ref,
                 w1uT_ref, w1iT_ref, b1_ref, w2T_ref, b2_ref, outT_ref):
    x1 = jnp.dot(w1uT_ref[...], ueT_ref[...],
                 preferred_element_type=jnp.float32)
    x2 = jnp.dot(w1iT_ref[...], ieT_ref[...],
                 preferred_element_type=jnp.float32)
    h = jnp.maximum(x1 + x2 + b1_ref[...], 0.0)
    t = jnp.dot(w2T_ref[...], h, preferred_element_type=jnp.float32)
    outT_ref[...] = t + b2_ref[...] + suT_ref[...] + siT_ref[...]


def _tc_mlp(ueT, ieT, suT, siT, w1uT, w1iT, b1c, w2T, b2c):
    blk = 2048
    grid = B // blk
    return pl.pallas_call(
        _tc_mlp_body,
        out_shape=jax.ShapeDtypeStruct((N_RATINGS, B), jnp.float32),
        grid=(grid,),
        in_specs=[
            pl.BlockSpec((N_DIM, blk), lambda i: (0, i)),
            pl.BlockSpec((N_DIM, blk), lambda i: (0, i)),
            pl.BlockSpec((N_RATINGS, blk), lambda i: (0, i)),
            pl.BlockSpec((N_RATINGS, blk), lambda i: (0, i)),
            pl.BlockSpec((3 * N_RATINGS, N_DIM), lambda i: (0, 0)),
            pl.BlockSpec((3 * N_RATINGS, N_DIM), lambda i: (0, 0)),
            pl.BlockSpec((3 * N_RATINGS, 1), lambda i: (0, 0)),
            pl.BlockSpec((N_RATINGS, 3 * N_RATINGS), lambda i: (0, 0)),
            pl.BlockSpec((N_RATINGS, 1), lambda i: (0, 0)),
        ],
        out_specs=pl.BlockSpec((N_RATINGS, blk), lambda i: (0, i)),
    )(ueT, ieT, suT, siT, w1uT, w1iT, b1c, w2T, b2c)


def kernel(users, items, u_emb, i_emb, u_intercept, i_intercept,
           W1, b1, W2, b2):
    ueT, ieT, suT, siT = _sc_gather(users, items, u_emb.T, i_emb.T,
                                    u_intercept.T, i_intercept.T)
    w1T = W1.T
    outT = _tc_mlp(ueT, ieT, suT, siT,
                   w1T[:, :N_DIM], w1T[:, N_DIM:],
                   b1.reshape(-1, 1), W2.T, b2.reshape(-1, 1))
    return outT.T


# speedup vs baseline: 4.3388x; 4.3388x over previous
import functools

import jax
import jax.numpy as jnp
from jax import lax
from jax.experimental import pallas as pl
from jax.experimental.pallas import tpu as pltpu
from jax.experimental.pallas import tpu_sc as plsc

N_DIM = 32
N_RATINGS = 5
B = 16384
N_USERS = 1000000
N_ITEMS = 100000

_NC = 2
_NS = 16
_NW = _NC * _NS
_BPW = B // _NW
_GSUB = 128
_CHUNK = 4096


def _pack_emb_body(inT_ref, out_ref):
    x = inT_ref[...]
    out_ref[...] = jnp.concatenate(
        [x[:, w * 1024:(w + 1) * 1024].T for w in range(4)], axis=1)


def _pack_int_body(inT_ref, out_ref):
    x = inT_ref[...]
    z = jnp.zeros((512, 16 - N_RATINGS), jnp.float32)
    out_ref[...] = jnp.concatenate(
        [jnp.concatenate([x[:, w * 512:(w + 1) * 512].T, z], axis=1)
         for w in range(8)], axis=1)


def _pack_emb(tT, n):
    g = (n + _CHUNK - 1) // _CHUNK
    return pl.pallas_call(
        _pack_emb_body,
        out_shape=jax.ShapeDtypeStruct((g * 1024, 128), jnp.float32),
        grid=(g,),
        in_specs=[pl.BlockSpec((N_DIM, _CHUNK), lambda i: (0, i))],
        out_specs=pl.BlockSpec((1024, 128), lambda i: (i, 0)),
    )(tT)


def _pack_int(tT, n):
    g = (n + _CHUNK - 1) // _CHUNK
    return pl.pallas_call(
        _pack_int_body,
        out_shape=jax.ShapeDtypeStruct((g * 512, 128), jnp.float32),
        grid=(g,),
        in_specs=[pl.BlockSpec((N_RATINGS, _CHUNK), lambda i: (0, i))],
        out_specs=pl.BlockSpec((512, 128), lambda i: (i, 0)),
    )(tT)


def _sc_gather_body(users_hbm, items_hbm, pu_hbm, pi_hbm, qu_hbm, qi_hbm,
                    rawu_out, rawi_out, raw5u_out, raw5i_out,
                    idxu_v, idxi_v, rowu_v, rowi_v, row5u_v, row5i_v,
                    gu_v, gi_v, g5u_v, g5i_v,
                    sem0, sem1, sem2, sem3):
    wid = lax.axis_index("s") * _NC + lax.axis_index("c")
    base = wid * _BPW
    pltpu.sync_copy(users_hbm.at[pl.ds(base, _BPW)], idxu_v)
    pltpu.sync_copy(items_hbm.at[pl.ds(base, _BPW)], idxi_v)
    for k in range(_BPW // 16):
        sl = pl.ds(16 * k, 16)
        iu = idxu_v[sl]
        ii = idxi_v[sl]
        rowu_v[sl] = ((iu >> 12) << 10) + (iu & 1023)
        rowi_v[sl] = ((ii >> 12) << 10) + (ii & 1023)
        row5u_v[sl] = ((iu >> 12) << 9) + (iu & 511)
        row5i_v[sl] = ((ii >> 12) << 9) + (ii & 511)
    for s in range(_BPW // _GSUB):
        sl = pl.ds(s * _GSUB, _GSUB)
        c0 = pltpu.async_copy(pu_hbm.at[rowu_v.at[sl]], gu_v, sem0)
        c1 = pltpu.async_copy(pi_hbm.at[rowi_v.at[sl]], gi_v, sem1)
        c2 = pltpu.async_copy(qu_hbm.at[row5u_v.at[sl]], g5u_v, sem2)
        c3 = pltpu.async_copy(qi_hbm.at[row5i_v.at[sl]], g5i_v, sem3)
        out_sl = pl.ds(base + s * _GSUB, _GSUB)
        c0.wait()
        pltpu.sync_copy(gu_v, rawu_out.at[out_sl, :])
        c1.wait()
        pltpu.sync_copy(gi_v, rawi_out.at[out_sl, :])
        c2.wait()
        pltpu.sync_copy(g5u_v, raw5u_out.at[out_sl, :])
        c3.wait()
        pltpu.sync_copy(g5i_v, raw5i_out.at[out_sl, :])


def _sc_gather(users, items, pu, pi, qu, qi):
    f = functools.partial(
        pl.kernel,
        out_type=(
            jax.ShapeDtypeStruct((B, 128), jnp.float32),
            jax.ShapeDtypeStruct((B, 128), jnp.float32),
            jax.ShapeDtypeStruct((B, 128), jnp.float32),
            jax.ShapeDtypeStruct((B, 128), jnp.float32),
        ),
        mesh=plsc.VectorSubcoreMesh(core_axis_name="c",
                                    subcore_axis_name="s"),
        compiler_params=pltpu.CompilerParams(use_tc_tiling_on_sc=False,
                                             needs_layout_passes=False),
        scratch_types=[
            pltpu.VMEM((_BPW,), jnp.int32),
            pltpu.VMEM((_BPW,), jnp.int32),
            pltpu.VMEM((_BPW,), jnp.int32),
            pltpu.VMEM((_BPW,), jnp.int32),
            pltpu.VMEM((_BPW,), jnp.int32),
            pltpu.VMEM((_BPW,), jnp.int32),
            pltpu.VMEM((_GSUB, 128), jnp.float32),
            pltpu.VMEM((_GSUB, 128), jnp.float32),
            pltpu.VMEM((_GSUB, 128), jnp.float32),
            pltpu.VMEM((_GSUB, 128), jnp.float32),
            pltpu.SemaphoreType.DMA,
            pltpu.SemaphoreType.DMA,
            pltpu.SemaphoreType.DMA,
            pltpu.SemaphoreType.DMA,
        ],
    )(_sc_gather_body)
    return f(users, items, pu, pi, qu, qi)


def _tc_mlp_body(rawu_ref, rawi_ref, raw5u_ref, raw5i_ref, u_ref, i_ref,
                 w1u_ref, w1i_ref, b1_ref, w2_ref, b2_ref, sel_ref,
                 out_ref):
    blk = rawu_ref.shape[0]
    col = lax.broadcasted_iota(jnp.int32, (blk, 128), 1)
    u = u_ref[0, :].reshape(blk, 1)
    i = i_ref[0, :].reshape(blk, 1)
    phu = (u >> 10) & 3
    phi = (i >> 10) & 3
    ph5u = (u >> 9) & 7
    ph5i = (i >> 9) & 7
    xu = jnp.where((col >> 5) == phu, rawu_ref[...], 0.0)
    xi = jnp.where((col >> 5) == phi, rawi_ref[...], 0.0)
    x5u = jnp.where((col >> 4) == ph5u, raw5u_ref[...], 0.0)
    x5i = jnp.where((col >> 4) == ph5i, raw5i_ref[...], 0.0)
    h = jnp.maximum(
        jnp.dot(xu, w1u_ref[...], preferred_element_type=jnp.float32)
        + jnp.dot(xi, w1i_ref[...], preferred_element_type=jnp.float32)
        + b1_ref[...], 0.0)
    t = jnp.dot(h, w2_ref[...], preferred_element_type=jnp.float32)
    su = jnp.dot(x5u, sel_ref[...], preferred_element_type=jnp.float32)
    si = jnp.dot(x5i, sel_ref[...], preferred_element_type=jnp.float32)
    out_ref[...] = t + b2_ref[...] + su + si


def _tc_mlp(rawu, rawi, raw5u, raw5i, u2, i2, w1u4, w1i4, b1r, w2, b2r,
            sel):
    blk = 2048
    grid = B // blk
    return pl.pallas_call(
        _tc_mlp_body,
        out_shape=jax.ShapeDtypeStruct((B, N_RATINGS), jnp.float32),
        grid=(grid,),
        in_specs=[
            pl.BlockSpec((blk, 128), lambda i: (i, 0)),
            pl.BlockSpec((blk, 128), lambda i: (i, 0)),
            pl.BlockSpec((blk, 128), lambda i: (i, 0)),
            pl.BlockSpec((blk, 128), lambda i: (i, 0)),
            pl.BlockSpec((1, blk), lambda i: (0, i)),
            pl.BlockSpec((1, blk), lambda i: (0, i)),
            pl.BlockSpec((128, 3 * N_RATINGS), lambda i: (0, 0)),
            pl.BlockSpec((128, 3 * N_RATINGS), lambda i: (0, 0)),
            pl.BlockSpec((1, 3 * N_RATINGS), lambda i: (0, 0)),
            pl.BlockSpec((3 * N_RATINGS, N_RATINGS), lambda i: (0, 0)),
            pl.BlockSpec((1, N_RATINGS), lambda i: (0, 0)),
            pl.BlockSpec((128, N_RATINGS), lambda i: (0, 0)),
        ],
        out_specs=pl.BlockSpec((blk, N_RATINGS), lambda i: (i, 0)),
    )(rawu, rawi, raw5u, raw5i, u2, i2, w1u4, w1i4, b1r, w2, b2r, sel)


def kernel(users, items, u_emb, i_emb, u_intercept, i_intercept,
           W1, b1, W2, b2):
    pu = _pack_emb(u_emb.T, N_USERS)
    pi = _pack_emb(i_emb.T, N_ITEMS)
    qu = _pack_int(u_intercept.T, N_USERS)
    qi = _pack_int(i_intercept.T, N_ITEMS)
    rawu, rawi, raw5u, raw5i = _sc_gather(users, items, pu, pi, qu, qi)
    w1u4 = jnp.tile(W1[:N_DIM], (4, 1))
    w1i4 = jnp.tile(W1[N_DIM:], (4, 1))
    sel = (jnp.arange(128)[:, None] % 16
           == jnp.arange(N_RATINGS)[None, :]).astype(jnp.float32)
    return _tc_mlp(rawu, rawi, raw5u, raw5i,
                   users.reshape(1, B), items.reshape(1, B),
                   w1u4, w1i4, b1.reshape(1, -1), W2, b2.reshape(1, -1),
                   sel)


# speedup vs baseline: 4.7414x; 1.0928x over previous
import functools

import jax
import jax.numpy as jnp
from jax import lax
from jax.experimental import pallas as pl
from jax.experimental.pallas import tpu as pltpu
from jax.experimental.pallas import tpu_sc as plsc

N_DIM = 32
N_RATINGS = 5
B = 16384
N_USERS = 1000000
N_ITEMS = 100000

_NC = 2
_NS = 16
_NW = _NC * _NS
_BPW = B // _NW
_GSUB = 128
_CHUNK = 16384


def _eye(n, m):
    return (lax.broadcasted_iota(jnp.int32, (n, m), 0)
            == lax.broadcasted_iota(jnp.int32, (n, m), 1)).astype(jnp.float32)


def _tdot(x, e):
    return lax.dot_general(x, e, (((0,), (0,)), ((), ())),
                           preferred_element_type=jnp.float32)


def _pack_emb_body(inT_ref, out_ref):
    x = inT_ref[...]
    p = _CHUNK // 4
    e = _eye(N_DIM, N_DIM)
    out_ref[...] = jnp.concatenate(
        [_tdot(x[:, w * p:(w + 1) * p], e) for w in range(4)], axis=1)


def _pack_int_body(inT_ref, out_ref):
    x = inT_ref[...]
    p = _CHUNK // 8
    e = _eye(N_RATINGS, 16)
    out_ref[...] = jnp.concatenate(
        [_tdot(x[:, w * p:(w + 1) * p], e) for w in range(8)], axis=1)


def _pack_emb(tT, n):
    g = (n + _CHUNK - 1) // _CHUNK
    return pl.pallas_call(
        _pack_emb_body,
        out_shape=jax.ShapeDtypeStruct((g * (_CHUNK // 4), 128),
                                       jnp.float32),
        grid=(g,),
        in_specs=[pl.BlockSpec((N_DIM, _CHUNK), lambda i: (0, i))],
        out_specs=pl.BlockSpec((_CHUNK // 4, 128), lambda i: (i, 0)),
    )(tT)


def _pack_int(tT, n):
    g = (n + _CHUNK - 1) // _CHUNK
    return pl.pallas_call(
        _pack_int_body,
        out_shape=jax.ShapeDtypeStruct((g * (_CHUNK // 8), 128),
                                       jnp.float32),
        grid=(g,),
        in_specs=[pl.BlockSpec((N_RATINGS, _CHUNK), lambda i: (0, i))],
        out_specs=pl.BlockSpec((_CHUNK // 8, 128), lambda i: (i, 0)),
    )(tT)


def _sc_gather_body(users_hbm, items_hbm, pu_hbm, pi_hbm, qu_hbm, qi_hbm,
                    rawu_out, rawi_out, raw5u_out, raw5i_out,
                    idxu_v, idxi_v, rowu_v, rowi_v, row5u_v, row5i_v,
                    gu_v, gi_v, g5u_v, g5i_v,
                    sem0, sem1, sem2, sem3):
    wid = lax.axis_index("s") * _NC + lax.axis_index("c")
    base = wid * _BPW
    pltpu.sync_copy(users_hbm.at[pl.ds(base, _BPW)], idxu_v)
    pltpu.sync_copy(items_hbm.at[pl.ds(base, _BPW)], idxi_v)
    for k in range(_BPW // 16):
        sl = pl.ds(16 * k, 16)
        iu = idxu_v[sl]
        ii = idxi_v[sl]
        rowu_v[sl] = ((iu >> 14) << 12) + (iu & 4095)
        rowi_v[sl] = ((ii >> 14) << 12) + (ii & 4095)
        row5u_v[sl] = ((iu >> 14) << 11) + (iu & 2047)
        row5i_v[sl] = ((ii >> 14) << 11) + (ii & 2047)
    for s in range(_BPW // _GSUB):
        sl = pl.ds(s * _GSUB, _GSUB)
        c0 = pltpu.async_copy(pu_hbm.at[rowu_v.at[sl]], gu_v, sem0)
        c1 = pltpu.async_copy(pi_hbm.at[rowi_v.at[sl]], gi_v, sem1)
        c2 = pltpu.async_copy(qu_hbm.at[row5u_v.at[sl]], g5u_v, sem2)
        c3 = pltpu.async_copy(qi_hbm.at[row5i_v.at[sl]], g5i_v, sem3)
        out_sl = pl.ds(base + s * _GSUB, _GSUB)
        c0.wait()
        pltpu.sync_copy(gu_v, rawu_out.at[out_sl, :])
        c1.wait()
        pltpu.sync_copy(gi_v, rawi_out.at[out_sl, :])
        c2.wait()
        pltpu.sync_copy(g5u_v, raw5u_out.at[out_sl, :])
        c3.wait()
        pltpu.sync_copy(g5i_v, raw5i_out.at[out_sl, :])


def _sc_gather(users, items, pu, pi, qu, qi):
    f = functools.partial(
        pl.kernel,
        out_type=(
            jax.ShapeDtypeStruct((B, 128), jnp.float32),
            jax.ShapeDtypeStruct((B, 128), jnp.float32),
            jax.ShapeDtypeStruct((B, 128), jnp.float32),
            jax.ShapeDtypeStruct((B, 128), jnp.float32),
        ),
        mesh=plsc.VectorSubcoreMesh(core_axis_name="c",
                                    subcore_axis_name="s"),
        compiler_params=pltpu.CompilerParams(use_tc_tiling_on_sc=False,
                                             needs_layout_passes=False),
        scratch_types=[
            pltpu.VMEM((_BPW,), jnp.int32),
            pltpu.VMEM((_BPW,), jnp.int32),
            pltpu.VMEM((_BPW,), jnp.int32),
            pltpu.VMEM((_BPW,), jnp.int32),
            pltpu.VMEM((_BPW,), jnp.int32),
            pltpu.VMEM((_BPW,), jnp.int32),
            pltpu.VMEM((_GSUB, 128), jnp.float32),
            pltpu.VMEM((_GSUB, 128), jnp.float32),
            pltpu.VMEM((_GSUB, 128), jnp.float32),
            pltpu.VMEM((_GSUB, 128), jnp.float32),
            pltpu.SemaphoreType.DMA,
            pltpu.SemaphoreType.DMA,
            pltpu.SemaphoreType.DMA,
            pltpu.SemaphoreType.DMA,
        ],
    )(_sc_gather_body)
    return f(users, items, pu, pi, qu, qi)


def _tc_mlp_body(rawu_ref, rawi_ref, raw5u_ref, raw5i_ref, u_ref, i_ref,
                 w1u_ref, w1i_ref, b1_ref, w2_ref, b2_ref, sel_ref,
                 out_ref):
    blk = rawu_ref.shape[0]
    col = lax.broadcasted_iota(jnp.int32, (blk, 128), 1)
    u = u_ref[0, :].reshape(blk, 1)
    i = i_ref[0, :].reshape(blk, 1)
    phu = (u >> 12) & 3
    phi = (i >> 12) & 3
    ph5u = (u >> 11) & 7
    ph5i = (i >> 11) & 7
    xu = jnp.where((col >> 5) == phu, rawu_ref[...], 0.0)
    xi = jnp.where((col >> 5) == phi, rawi_ref[...], 0.0)
    x5u = jnp.where((col >> 4) == ph5u, raw5u_ref[...], 0.0)
    x5i = jnp.where((col >> 4) == ph5i, raw5i_ref[...], 0.0)
    h = jnp.maximum(
        jnp.dot(xu, w1u_ref[...], preferred_element_type=jnp.float32)
        + jnp.dot(xi, w1i_ref[...], preferred_element_type=jnp.float32)
        + b1_ref[...], 0.0)
    t = jnp.dot(h, w2_ref[...], preferred_element_type=jnp.float32)
    su = jnp.dot(x5u, sel_ref[...], preferred_element_type=jnp.float32)
    si = jnp.dot(x5i, sel_ref[...], preferred_element_type=jnp.float32)
    out_ref[...] = t + b2_ref[...] + su + si


def _tc_mlp(rawu, rawi, raw5u, raw5i, u2, i2, w1u4, w1i4, b1r, w2, b2r,
            sel):
    blk = 2048
    grid = B // blk
    return pl.pallas_call(
        _tc_mlp_body,
        out_shape=jax.ShapeDtypeStruct((B, N_RATINGS), jnp.float32),
        grid=(grid,),
        in_specs=[
            pl.BlockSpec((blk, 128), lambda i: (i, 0)),
            pl.BlockSpec((blk, 128), lambda i: (i, 0)),
            pl.BlockSpec((blk, 128), lambda i: (i, 0)),
            pl.BlockSpec((blk, 128), lambda i: (i, 0)),
            pl.BlockSpec((1, blk), lambda i: (0, i)),
            pl.BlockSpec((1, blk), lambda i: (0, i)),
            pl.BlockSpec((128, 3 * N_RATINGS), lambda i: (0, 0)),
            pl.BlockSpec((128, 3 * N_RATINGS), lambda i: (0, 0)),
            pl.BlockSpec((1, 3 * N_RATINGS), lambda i: (0, 0)),
            pl.BlockSpec((3 * N_RATINGS, N_RATINGS), lambda i: (0, 0)),
            pl.BlockSpec((1, N_RATINGS), lambda i: (0, 0)),
            pl.BlockSpec((128, N_RATINGS), lambda i: (0, 0)),
        ],
        out_specs=pl.BlockSpec((blk, N_RATINGS), lambda i: (i, 0)),
    )(rawu, rawi, raw5u, raw5i, u2, i2, w1u4, w1i4, b1r, w2, b2r, sel)


def kernel(users, items, u_emb, i_emb, u_intercept, i_intercept,
           W1, b1, W2, b2):
    pu = _pack_emb(u_emb.T, N_USERS)
    pi = _pack_emb(i_emb.T, N_ITEMS)
    qu = _pack_int(u_intercept.T, N_USERS)
    qi = _pack_int(i_intercept.T, N_ITEMS)
    rawu, rawi, raw5u, raw5i = _sc_gather(users, items, pu, pi, qu, qi)
    w1u4 = jnp.tile(W1[:N_DIM], (4, 1))
    w1i4 = jnp.tile(W1[N_DIM:], (4, 1))
    sel = (jnp.arange(128)[:, None] % 16
           == jnp.arange(N_RATINGS)[None, :]).astype(jnp.float32)
    return _tc_mlp(rawu, rawi, raw5u, raw5i,
                   users.reshape(1, B), items.reshape(1, B),
                   w1u4, w1i4, b1.reshape(1, -1), W2, b2.reshape(1, -1),
                   sel)


# speedup vs baseline: 7.1733x; 1.5129x over previous
import functools

import jax
import jax.numpy as jnp
from jax import lax
from jax.experimental import pallas as pl
from jax.experimental.pallas import tpu as pltpu
from jax.experimental.pallas import tpu_sc as plsc

N_DIM = 32
N_RATINGS = 5
N_HID = 15
B = 16384
N_USERS = 1000000
N_ITEMS = 100000

_NC = 2
_NS = 16
_NW = _NC * _NS
_BPW = B // _NW
_GSUB = 256
_CHUNK = 16384
_P = _CHUNK // 4


def _tdot(x, e):
    return lax.dot_general(x, e, (((0,), (0,)), ((), ())),
                           preferred_element_type=jnp.float32)


def _pack_body(embT_ref, intT_ref, w1x_ref, bsel_ref, out_ref):
    xe = embT_ref[...]
    xi = intT_ref[...]
    w1x = w1x_ref[...]
    bs = bsel_ref[...]
    pieces = []
    for w in range(4):
        sl = slice(w * _P, (w + 1) * _P)
        y = _tdot(xe[:, sl], w1x) + _tdot(xi[:, sl], bs)
        pieces.append(pltpu.bitcast(y.astype(jnp.bfloat16), jnp.float32))
    out_ref[...] = jnp.concatenate(pieces, axis=1)


def _pack(embT, intT, w1x, bsel, n):
    g = (n + _CHUNK - 1) // _CHUNK
    return pl.pallas_call(
        _pack_body,
        out_shape=jax.ShapeDtypeStruct((g * (_P // 2), 128), jnp.float32),
        grid=(g,),
        in_specs=[
            pl.BlockSpec((N_DIM, _CHUNK), lambda i: (0, i)),
            pl.BlockSpec((N_RATINGS, _CHUNK), lambda i: (0, i)),
            pl.BlockSpec((N_DIM, 32), lambda i: (0, 0)),
            pl.BlockSpec((N_RATINGS, 32), lambda i: (0, 0)),
        ],
        out_specs=pl.BlockSpec((_P // 2, 128), lambda i: (i, 0)),
    )(embT, intT, w1x, bsel)


def _sc_gather_body(users_hbm, items_hbm, pu_hbm, pi_hbm,
                    rawu_out, rawi_out,
                    idxu_v, idxi_v, rowu_v, rowi_v, gu_v, gi_v,
                    sem0, sem1):
    wid = lax.axis_index("s") * _NC + lax.axis_index("c")
    base = wid * _BPW
    pltpu.sync_copy(users_hbm.at[pl.ds(base, _BPW)], idxu_v)
    pltpu.sync_copy(items_hbm.at[pl.ds(base, _BPW)], idxi_v)
    for k in range(_BPW // 16):
        sl = pl.ds(16 * k, 16)
        iu = idxu_v[sl]
        ii = idxi_v[sl]
        rowu_v[sl] = ((iu >> 14) << 11) + ((iu & 4095) >> 1)
        rowi_v[sl] = ((ii >> 14) << 11) + ((ii & 4095) >> 1)
    for s in range(_BPW // _GSUB):
        sl = pl.ds(s * _GSUB, _GSUB)
        c0 = pltpu.async_copy(pu_hbm.at[rowu_v.at[sl]], gu_v, sem0)
        c1 = pltpu.async_copy(pi_hbm.at[rowi_v.at[sl]], gi_v, sem1)
        out_sl = pl.ds(base + s * _GSUB, _GSUB)
        c0.wait()
        pltpu.sync_copy(gu_v, rawu_out.at[out_sl, :])
        c1.wait()
        pltpu.sync_copy(gi_v, rawi_out.at[out_sl, :])


def _sc_gather(users, items, pu, pi):
    f = functools.partial(
        pl.kernel,
        out_type=(
            jax.ShapeDtypeStruct((B, 128), jnp.float32),
            jax.ShapeDtypeStruct((B, 128), jnp.float32),
        ),
        mesh=plsc.VectorSubcoreMesh(core_axis_name="c",
                                    subcore_axis_name="s"),
        compiler_params=pltpu.CompilerParams(use_tc_tiling_on_sc=False,
                                             needs_layout_passes=False),
        scratch_types=[
            pltpu.VMEM((_BPW,), jnp.int32),
            pltpu.VMEM((_BPW,), jnp.int32),
            pltpu.VMEM((_BPW,), jnp.int32),
            pltpu.VMEM((_BPW,), jnp.int32),
            pltpu.VMEM((_GSUB, 128), jnp.float32),
            pltpu.VMEM((_GSUB, 128), jnp.float32),
            pltpu.SemaphoreType.DMA,
            pltpu.SemaphoreType.DMA,
        ],
    )(_sc_gather_body)
    return f(users, items, pu, pi)


def _tc_mlp_body(rawu_ref, rawi_ref, u_ref, i_ref,
                 b1_ref, w2_ref, b2_ref, out_ref):
    blk = rawu_ref.shape[0]
    vu = lax.bitcast_convert_type(rawu_ref[...], jnp.int32)
    vi = lax.bitcast_convert_type(rawi_ref[...], jnp.int32)
    col = lax.broadcasted_iota(jnp.int32, (blk, 128), 1)
    u = u_ref[0, :].reshape(blk, 1)
    i = i_ref[0, :].reshape(blk, 1)
    hi_mask = jnp.int32(-65536)
    bu = jnp.where((u & 1) == 1, vu & hi_mask, vu << 16)
    bi = jnp.where((i & 1) == 1, vi & hi_mask, vi << 16)
    xu = lax.bitcast_convert_type(bu, jnp.float32)
    xi = lax.bitcast_convert_type(bi, jnp.float32)
    phu = (u >> 12) & 3
    phi = (i >> 12) & 3
    xu = jnp.where((col >> 5) == phu, xu, 0.0)
    xi = jnp.where((col >> 5) == phi, xi, 0.0)
    r = lax.broadcasted_iota(jnp.int32, (128, N_HID), 1)
    sel_h = (lax.broadcasted_iota(jnp.int32, (128, N_HID), 0) % 32
             == r).astype(jnp.float32)
    j = lax.broadcasted_iota(jnp.int32, (128, N_RATINGS), 1)
    sel_s = (lax.broadcasted_iota(jnp.int32, (128, N_RATINGS), 0) % 32
             == 16 + j).astype(jnp.float32)
    hu = jnp.dot(xu, sel_h, preferred_element_type=jnp.float32)
    hi = jnp.dot(xi, sel_h, preferred_element_type=jnp.float32)
    su = jnp.dot(xu, sel_s, preferred_element_type=jnp.float32)
    si = jnp.dot(xi, sel_s, preferred_element_type=jnp.float32)
    h = jnp.maximum(hu + hi + b1_ref[...], 0.0)
    t = jnp.dot(h, w2_ref[...], preferred_element_type=jnp.float32)
    out_ref[...] = t + b2_ref[...] + su + si


def _tc_mlp(rawu, rawi, u2, i2, b1r, w2, b2r):
    blk = 2048
    grid = B // blk
    return pl.pallas_call(
        _tc_mlp_body,
        out_shape=jax.ShapeDtypeStruct((B, N_RATINGS), jnp.float32),
        grid=(grid,),
        in_specs=[
            pl.BlockSpec((blk, 128), lambda i: (i, 0)),
            pl.BlockSpec((blk, 128), lambda i: (i, 0)),
            pl.BlockSpec((1, blk), lambda i: (0, i)),
            pl.BlockSpec((1, blk), lambda i: (0, i)),
            pl.BlockSpec((1, N_HID), lambda i: (0, 0)),
            pl.BlockSpec((N_HID, N_RATINGS), lambda i: (0, 0)),
            pl.BlockSpec((1, N_RATINGS), lambda i: (0, 0)),
        ],
        out_specs=pl.BlockSpec((blk, N_RATINGS), lambda i: (i, 0)),
    )(rawu, rawi, u2, i2, b1r, w2, b2r)


def kernel(users, items, u_emb, i_emb, u_intercept, i_intercept,
           W1, b1, W2, b2):
    zpad = jnp.zeros((N_DIM, 32 - N_HID), jnp.float32)
    w1xu = jnp.concatenate([W1[:N_DIM], zpad], axis=1)
    w1xi = jnp.concatenate([W1[N_DIM:], zpad], axis=1)
    eye5 = jnp.eye(N_RATINGS, dtype=jnp.float32)
    bsel = jnp.concatenate(
        [jnp.zeros((N_RATINGS, 16), jnp.float32), eye5,
         jnp.zeros((N_RATINGS, 32 - 16 - N_RATINGS), jnp.float32)],
        axis=1)
    pu = _pack(u_emb.T, u_intercept.T, w1xu, bsel, N_USERS)
    pi = _pack(i_emb.T, i_intercept.T, w1xi, bsel, N_ITEMS)
    rawu, rawi = _sc_gather(users, items, pu, pi)
    return _tc_mlp(rawu, rawi, users.reshape(1, B), items.reshape(1, B),
                   b1.reshape(1, -1), W2, b2.reshape(1, -1))


# speedup vs baseline: 10.9286x; 1.5235x over previous
import functools

import jax
import jax.numpy as jnp
from jax import lax
from jax.experimental import pallas as pl
from jax.experimental.pallas import tpu as pltpu
from jax.experimental.pallas import tpu_sc as plsc

N_DIM = 32
N_RATINGS = 5
N_HID = 15
B = 16384
N_USERS = 1000000
N_ITEMS = 100000

_NC = 2
_NS = 16
_NW = _NC * _NS
_BPW = B // _NW
_GSUB = 256
_CHUNK = 16384
_P = _CHUNK // 4


def _tdot(x, e):
    return lax.dot_general(x, e, (((0,), (0,)), ((), ())),
                           preferred_element_type=jnp.float32)


def _pack_body(embT_ref, intT_ref, w1x_ref, bsel_ref, out_ref):
    xe = embT_ref[...].astype(jnp.bfloat16)
    xi = intT_ref[...].astype(jnp.bfloat16)
    w1x = w1x_ref[...].astype(jnp.bfloat16)
    bs = bsel_ref[...].astype(jnp.bfloat16)
    pieces = []
    for w in range(4):
        sl = slice(w * _P, (w + 1) * _P)
        y = _tdot(xe[:, sl], w1x) + _tdot(xi[:, sl], bs)
        pieces.append(pltpu.bitcast(y.astype(jnp.bfloat16), jnp.float32))
    out_ref[...] = jnp.concatenate(pieces, axis=1)


def _pack(embT, intT, w1x, bsel, n):
    g = (n + _CHUNK - 1) // _CHUNK
    return pl.pallas_call(
        _pack_body,
        out_shape=jax.ShapeDtypeStruct((g * (_P // 2), 128), jnp.float32),
        grid=(g,),
        in_specs=[
            pl.BlockSpec((N_DIM, _CHUNK), lambda i: (0, i)),
            pl.BlockSpec((N_RATINGS, _CHUNK), lambda i: (0, i)),
            pl.BlockSpec((N_DIM, 32), lambda i: (0, 0)),
            pl.BlockSpec((N_RATINGS, 32), lambda i: (0, 0)),
        ],
        out_specs=pl.BlockSpec((_P // 2, 128), lambda i: (i, 0)),
    )(embT, intT, w1x, bsel)


def _sc_gather_body(users_hbm, items_hbm, pu_hbm, pi_hbm,
                    rawu_out, rawi_out,
                    idxu_v, idxi_v, rowu_v, rowi_v, gu_v, gi_v,
                    sem0, sem1):
    wid = lax.axis_index("s") * _NC + lax.axis_index("c")
    base = wid * _BPW
    pltpu.sync_copy(users_hbm.at[pl.ds(base, _BPW)], idxu_v)
    pltpu.sync_copy(items_hbm.at[pl.ds(base, _BPW)], idxi_v)
    for k in range(_BPW // 16):
        sl = pl.ds(16 * k, 16)
        iu = idxu_v[sl]
        ii = idxi_v[sl]
        rowu_v[sl] = ((iu >> 14) << 11) + ((iu & 4095) >> 1)
        rowi_v[sl] = ((ii >> 14) << 11) + ((ii & 4095) >> 1)
    for s in range(_BPW // _GSUB):
        sl = pl.ds(s * _GSUB, _GSUB)
        c0 = pltpu.async_copy(pu_hbm.at[rowu_v.at[sl]], gu_v, sem0)
        c1 = pltpu.async_copy(pi_hbm.at[rowi_v.at[sl]], gi_v, sem1)
        out_sl = pl.ds(base + s * _GSUB, _GSUB)
        c0.wait()
        pltpu.sync_copy(gu_v, rawu_out.at[out_sl, :])
        c1.wait()
        pltpu.sync_copy(gi_v, rawi_out.at[out_sl, :])


def _sc_gather(users, items, pu, pi):
    f = functools.partial(
        pl.kernel,
        out_type=(
            jax.ShapeDtypeStruct((B, 128), jnp.float32),
            jax.ShapeDtypeStruct((B, 128), jnp.float32),
        ),
        mesh=plsc.VectorSubcoreMesh(core_axis_name="c",
                                    subcore_axis_name="s"),
        compiler_params=pltpu.CompilerParams(use_tc_tiling_on_sc=False,
                                             needs_layout_passes=False),
        scratch_types=[
            pltpu.VMEM((_BPW,), jnp.int32),
            pltpu.VMEM((_BPW,), jnp.int32),
            pltpu.VMEM((_BPW,), jnp.int32),
            pltpu.VMEM((_BPW,), jnp.int32),
            pltpu.VMEM((_GSUB, 128), jnp.float32),
            pltpu.VMEM((_GSUB, 128), jnp.float32),
            pltpu.SemaphoreType.DMA,
            pltpu.SemaphoreType.DMA,
        ],
    )(_sc_gather_body)
    return f(users, items, pu, pi)


def _tc_mlp_body(rawu_ref, rawi_ref, u_ref, i_ref,
                 b1_ref, w2_ref, b2_ref, out_ref):
    blk = rawu_ref.shape[0]
    vu = lax.bitcast_convert_type(rawu_ref[...], jnp.int32)
    vi = lax.bitcast_convert_type(rawi_ref[...], jnp.int32)
    col = lax.broadcasted_iota(jnp.int32, (blk, 128), 1)
    u = u_ref[0, :].reshape(blk, 1)
    i = i_ref[0, :].reshape(blk, 1)
    hi_mask = jnp.int32(-65536)
    bu = jnp.where((u & 1) == 1, vu & hi_mask, vu << 16)
    bi = jnp.where((i & 1) == 1, vi & hi_mask, vi << 16)
    xu = lax.bitcast_convert_type(bu, jnp.float32)
    xi = lax.bitcast_convert_type(bi, jnp.float32)
    phu = (u >> 12) & 3
    phi = (i >> 12) & 3
    xu = jnp.where((col >> 5) == phu, xu, 0.0)
    xi = jnp.where((col >> 5) == phi, xi, 0.0)
    r = lax.broadcasted_iota(jnp.int32, (128, N_HID), 1)
    sel_h = (lax.broadcasted_iota(jnp.int32, (128, N_HID), 0) % 32
             == r).astype(jnp.float32)
    j = lax.broadcasted_iota(jnp.int32, (128, N_RATINGS), 1)
    sel_s = (lax.broadcasted_iota(jnp.int32, (128, N_RATINGS), 0) % 32
             == 16 + j).astype(jnp.float32)
    hu = jnp.dot(xu, sel_h, preferred_element_type=jnp.float32)
    hi = jnp.dot(xi, sel_h, preferred_element_type=jnp.float32)
    su = jnp.dot(xu, sel_s, preferred_element_type=jnp.float32)
    si = jnp.dot(xi, sel_s, preferred_element_type=jnp.float32)
    h = jnp.maximum(hu + hi + b1_ref[...], 0.0)
    t = jnp.dot(h, w2_ref[...], preferred_element_type=jnp.float32)
    out_ref[...] = t + b2_ref[...] + su + si


def _tc_mlp(rawu, rawi, u2, i2, b1r, w2, b2r):
    blk = 2048
    grid = B // blk
    return pl.pallas_call(
        _tc_mlp_body,
        out_shape=jax.ShapeDtypeStruct((B, N_RATINGS), jnp.float32),
        grid=(grid,),
        in_specs=[
            pl.BlockSpec((blk, 128), lambda i: (i, 0)),
            pl.BlockSpec((blk, 128), lambda i: (i, 0)),
            pl.BlockSpec((1, blk), lambda i: (0, i)),
            pl.BlockSpec((1, blk), lambda i: (0, i)),
            pl.BlockSpec((1, N_HID), lambda i: (0, 0)),
            pl.BlockSpec((N_HID, N_RATINGS), lambda i: (0, 0)),
            pl.BlockSpec((1, N_RATINGS), lambda i: (0, 0)),
        ],
        out_specs=pl.BlockSpec((blk, N_RATINGS), lambda i: (i, 0)),
    )(rawu, rawi, u2, i2, b1r, w2, b2r)


def kernel(users, items, u_emb, i_emb, u_intercept, i_intercept,
           W1, b1, W2, b2):
    zpad = jnp.zeros((N_DIM, 32 - N_HID), jnp.float32)
    w1xu = jnp.concatenate([W1[:N_DIM], zpad], axis=1)
    w1xi = jnp.concatenate([W1[N_DIM:], zpad], axis=1)
    eye5 = jnp.eye(N_RATINGS, dtype=jnp.float32)
    bsel = jnp.concatenate(
        [jnp.zeros((N_RATINGS, 16), jnp.float32), eye5,
         jnp.zeros((N_RATINGS, 32 - 16 - N_RATINGS), jnp.float32)],
        axis=1)
    pu = _pack(u_emb.T, u_intercept.T, w1xu, bsel, N_USERS)
    pi = _pack(i_emb.T, i_intercept.T, w1xi, bsel, N_ITEMS)
    rawu, rawi = _sc_gather(users, items, pu, pi)
    return _tc_mlp(rawu, rawi, users.reshape(1, B), items.reshape(1, B),
                   b1.reshape(1, -1), W2, b2.reshape(1, -1))
